# baseline (speedup 1.0000x reference)
import jax
import jax.numpy as jnp
from jax import lax
from jax.experimental import pallas as pl
from jax.experimental.pallas import tpu as pltpu
from jax.experimental.pallas import tpu_sc as plsc


def kernel(kx, vx, k_cache, v_cache):
    B, _, D = kx.shape
    total = B * D
    half = total // 2
    kx1 = kx.reshape(total)
    vx1 = vx.reshape(total)

    mesh = plsc.VectorSubcoreMesh(core_axis_name="c", subcore_axis_name="s", num_cores=1)

    chunk = total // 16

    def body(kx_hbm, vx_hbm, ko_hbm, vo_hbm, sem):
        s_ = lax.axis_index("s")
        base = s_ * chunk
        ck = pltpu.make_async_copy(kx_hbm.at[pl.ds(base, chunk)], ko_hbm.at[pl.ds(base, chunk)], sem)
        cv = pltpu.make_async_copy(vx_hbm.at[pl.ds(base, chunk)], vo_hbm.at[pl.ds(base, chunk)], sem)
        ck.start()
        cv.start()
        ck.wait()
        cv.wait()

    out_k, out_v = pl.kernel(
        body,
        mesh=mesh,
        out_type=(
            jax.ShapeDtypeStruct((total,), kx.dtype),
            jax.ShapeDtypeStruct((total,), vx.dtype),
        ),
        scratch_types=[pltpu.SemaphoreType.DMA],
    )(kx1, vx1)

    return (out_k.reshape(B, 1, D), out_v.reshape(B, 1, D))


# speedup vs baseline: 1.0707x; 1.0707x over previous
import jax
from jax.experimental import pallas as pl
from jax.experimental.pallas import tpu as pltpu
from jax.experimental.pallas import tpu_sc as plsc


def kernel(kx, vx, k_cache, v_cache):
    B, _, D = kx.shape
    total = B * D
    kx1 = kx.reshape(total)
    vx1 = vx.reshape(total)

    mesh = plsc.ScalarSubcoreMesh(axis_name="c", num_cores=1)

    def body(kx_hbm, vx_hbm, ko_hbm, vo_hbm, sem):
        ck = pltpu.make_async_copy(kx_hbm, ko_hbm, sem)
        cv = pltpu.make_async_copy(vx_hbm, vo_hbm, sem)
        ck.start()
        cv.start()
        ck.wait()
        cv.wait()

    out_k, out_v = pl.kernel(
        body,
        mesh=mesh,
        out_type=(
            jax.ShapeDtypeStruct((total,), kx.dtype),
            jax.ShapeDtypeStruct((total,), vx.dtype),
        ),
        scratch_types=[pltpu.SemaphoreType.DMA],
    )(kx1, vx1)

    return (out_k.reshape(B, 1, D), out_v.reshape(B, 1, D))
